# baseline (speedup 1.0000x reference)
import jax
import jax.numpy as jnp
from jax import lax
from jax.experimental import pallas as pl
from jax.experimental.pallas import tpu as pltpu
from jax.experimental.pallas import tpu_sc as plsc

N = 10000
D = 128
H = 32
E = 160000
SD = 9
NROT = 2000
K = 16
ET = NROT * K
NRBF = 32
DMAX = 5.0
NS = 128
SQ3 = 3.0 ** 0.5
SQ5 = 5.0 ** 0.5
SQ15 = 15.0 ** 0.5

NC = 2
NSUB = 16
NW = NC * NSUB

EP = 163840
ETP = 32768
NROTP = 2048
NUP = 12288
ACC = 10112


def _sc_mesh():
    return plsc.VectorSubcoreMesh(
        core_axis_name="c", subcore_axis_name="s",
        num_cores=NC, num_subcores=NSUB)


F0 = 0.69


def _core_split(nch_total):
    per_pair = nch_total // NSUB
    cpt0 = max(1, min(per_pair - 1, round(per_pair * F0)))
    return cpt0, per_pair - cpt0


def _sc_gather(table, idx, chunk=128):
    B = idx.shape[0]
    Dc = table.shape[1]
    dtype = table.dtype
    nch_total = B // chunk
    assert B % (NSUB * chunk) == 0
    cpt0, cpt1 = _core_split(nch_total)
    cptm = max(cpt0, cpt1)

    NB = max(1, min(4, cpt0, (131000 - cptm * chunk) // (chunk * Dc)))

    def body(table_hbm, idx_hbm, out_hbm, idx_v, *rest):
        bufs = rest[:NB]
        gsems = rest[NB:2 * NB]
        ssems = rest[2 * NB:3 * NB]
        c = lax.axis_index("c")
        s = lax.axis_index("s")

        def ring(cbase_fn, nchunks):
            base = cbase_fn(s) * chunk
            pltpu.sync_copy(idx_hbm.at[pl.ds(base, nchunks * chunk)],
                            idx_v.at[pl.ds(0, nchunks * chunk)])
            gd = {}
            sd = {}

            def start_gather(j):
                b = j % NB
                gd[j] = pltpu.async_copy(
                    table_hbm.at[idx_v.at[pl.ds(j * chunk, chunk)]],
                    bufs[b], gsems[b])

            for j in range(min(NB, nchunks)):
                start_gather(j)
            for j in range(nchunks):
                b = j % NB
                if j >= 1 and (j - 1 + NB) < nchunks:
                    sd[j - 1].wait()
                    start_gather(j - 1 + NB)
                gd[j].wait()
                sd[j] = pltpu.async_copy(
                    bufs[b], out_hbm.at[pl.ds(base + j * chunk, chunk)],
                    ssems[b])
            for t in range(max(0, nchunks - NB), nchunks):
                sd[t].wait()

        @pl.when(c == 0)
        def _():
            ring(lambda s_: s_ * cpt0, cpt0)

        @pl.when(c == 1)
        def _():
            ring(lambda s_: NSUB * cpt0 + s_ * cpt1, cpt1)

    fn = pl.kernel(
        body,
        out_type=jax.ShapeDtypeStruct((B, Dc), dtype),
        mesh=_sc_mesh(),
        scratch_types=(
            [pltpu.VMEM((cptm * chunk,), jnp.int32)]
            + [pltpu.VMEM((chunk, Dc), dtype)] * NB
            + [pltpu.SemaphoreType.DMA] * (2 * NB)
        ))
    return fn(table, idx)


def _sc_scatter_add(vals, idx2d, zeros_chunk):
    B, Dc = vals.shape
    chunk = 128
    nch_total = B // chunk
    zrows = ACC // NSUB
    assert zrows * NSUB == ACC
    per_pair = nch_total // NSUB
    if per_pair >= 16 and per_pair % 8 == 0:
        cpt0 = min(per_pair - 8, max(8, round(per_pair * F0 / 8) * 8))
    else:
        cpt0 = per_pair - per_pair // 2
    cpt1 = per_pair - cpt0
    cptm = max(cpt0, cpt1)

    NB = min(2, cpt0)

    def body(vals_hbm, idx_hbm, zeros_hbm, out_hbm, idx_v, acc, *rest):
        bufs = rest[:NB]
        lsems = rest[NB:2 * NB]
        asems = rest[2 * NB:3 * NB]
        c = lax.axis_index("c")
        s = lax.axis_index("s")

        pltpu.sync_copy(zeros_hbm, bufs[0])
        zbase = s * zrows
        off = 0
        while off < zrows:
            sz = min(chunk, zrows - off)
            pltpu.sync_copy(bufs[0].at[pl.ds(0, sz)],
                            acc.at[pl.ds(zbase + off, sz)])
            off += sz
        plsc.subcore_barrier()

        def ring(rowbase, nchunks):
            pltpu.sync_copy(idx_hbm.at[pl.ds(rowbase, nchunks)],
                            idx_v.at[pl.ds(0, nchunks)])
            vbase = rowbase * chunk
            ld = {}
            ad = {}

            def start_load(j):
                b = j % NB
                ld[j] = pltpu.async_copy(
                    vals_hbm.at[pl.ds(vbase + j * chunk, chunk)],
                    bufs[b], lsems[b])

            for j in range(min(NB, nchunks)):
                start_load(j)
            for j in range(nchunks):
                b = j % NB
                if j >= 1 and (j - 1 + NB) < nchunks:
                    ad[j - 1].wait()
                    start_load(j - 1 + NB)
                ld[j].wait()
                ad[j] = pltpu.async_copy(bufs[b], acc.at[idx_v.at[j]],
                                         asems[b], add=True)
            for t in range(max(0, nchunks - NB), nchunks):
                ad[t].wait()

        @pl.when(c == 0)
        def _():
            ring(s * cpt0, cpt0)

        @pl.when(c == 1)
        def _():
            ring(NSUB * cpt0 + s * cpt1, cpt1)

        plsc.subcore_barrier()

        off = 0
        while off < zrows:
            sz = min(chunk, zrows - off)
            pltpu.sync_copy(acc.at[pl.ds(zbase + off, sz)],
                            bufs[0].at[pl.ds(0, sz)])
            pltpu.sync_copy(bufs[0].at[pl.ds(0, sz)],
                            out_hbm.at[c, pl.ds(zbase + off, sz)])
            off += sz

    fn = pl.kernel(
        body,
        out_type=jax.ShapeDtypeStruct((NC, ACC, Dc), jnp.float32),
        mesh=_sc_mesh(),
        scratch_types=(
            [pltpu.VMEM((cptm, 128), jnp.int32),
             pltpu.VMEM_SHARED((ACC, Dc), jnp.float32)]
            + [pltpu.VMEM((chunk, Dc), jnp.float32)] * NB
            + [pltpu.SemaphoreType.DMA] * (2 * NB)
        ))
    return fn(vals, idx2d, zeros_chunk)


def _dot(a, b):
    return jax.lax.dot_general(a, b, (((1,), (0,)), ((), ())),
                               preferred_element_type=jnp.float32)


def _dot_t(a, b):
    return jax.lax.dot_general(a, b, (((1,), (1,)), ((), ())),
                               preferred_element_type=jnp.float32)


def _mm(x, w, bn=1000):
    M, Kc = x.shape
    Do = w.shape[1]
    grid = M // bn
    assert M % bn == 0

    def kern(xr, wr, o):
        o[...] = _dot(xr[...], wr[...])

    return pl.pallas_call(
        kern, grid=(grid,),
        in_specs=[pl.BlockSpec((bn, Kc), lambda i: (i, 0)),
                  pl.BlockSpec((Kc, Do), lambda i: (0, 0))],
        out_specs=pl.BlockSpec((bn, Do), lambda i: (i, 0)),
        out_shape=jax.ShapeDtypeStruct((M, Do), jnp.float32))(x, w)


def _edge_dense(g, ef, esh, w1, b1, w2, b2, msh, bn=1024):
    M, Dg = g.shape
    F = ef.shape[1]
    Se = esh.shape[1]
    Hh = w1.shape[1]
    grid = M // bn

    def kern(gr, efr, eshr, w1r, b1r, w2r, b2r, mshr, outr):
        h = jnp.maximum(_dot(efr[...], w1r[...]) + b1r[...], 0.0)
        w = _dot(h, w2r[...]) + b2r[...]
        outr[...] = gr[...] * w * _dot(eshr[...], mshr[...])

    return pl.pallas_call(
        kern, grid=(grid,),
        in_specs=[
            pl.BlockSpec((bn, Dg), lambda i: (i, 0)),
            pl.BlockSpec((bn, F), lambda i: (i, 0)),
            pl.BlockSpec((bn, Se), lambda i: (i, 0)),
            pl.BlockSpec((F, Hh), lambda i: (0, 0)),
            pl.BlockSpec((1, Hh), lambda i: (0, 0)),
            pl.BlockSpec((Hh, Dg), lambda i: (0, 0)),
            pl.BlockSpec((1, Dg), lambda i: (0, 0)),
            pl.BlockSpec((Se, Dg), lambda i: (0, 0)),
        ],
        out_specs=pl.BlockSpec((bn, Dg), lambda i: (i, 0)),
        out_shape=jax.ShapeDtypeStruct((M, Dg), jnp.float32))(
            g, ef, esh, w1, b1, w2, b2, msh)


def _edge_dense_tor(gxp, ea, gx1, xrep, gxrep, tesh,
                    w1a, w1b, w1c, b1, w2, b2, msh, bn=512):
    M, Dg = gxp.shape
    Hh = w1a.shape[1]
    grid = M // bn

    def kern(gr, ear, g1r, xrr, gxrr, teshr,
             w1ar, w1br, w1cr, b1r, w2r, b2r, mshr, outr):
        h = (_dot(ear[...], w1ar[...]) + _dot(g1r[...], w1br[...])
             + _dot(xrr[...] + gxrr[...], w1cr[...]) + b1r[...])
        h = jnp.maximum(h, 0.0)
        w = _dot(h, w2r[...]) + b2r[...]
        outr[...] = gr[...] * w * _dot(teshr[...], mshr[...])

    return pl.pallas_call(
        kern, grid=(grid,),
        in_specs=[
            pl.BlockSpec((bn, Dg), lambda i: (i, 0)),
            pl.BlockSpec((bn, H), lambda i: (i, 0)),
            pl.BlockSpec((bn, D), lambda i: (i, 0)),
            pl.BlockSpec((bn, D), lambda i: (i, 0)),
            pl.BlockSpec((bn, D), lambda i: (i, 0)),
            pl.BlockSpec((bn, 128), lambda i: (i, 0)),
            pl.BlockSpec((H, Hh), lambda i: (0, 0)),
            pl.BlockSpec((D, Hh), lambda i: (0, 0)),
            pl.BlockSpec((D, Hh), lambda i: (0, 0)),
            pl.BlockSpec((1, Hh), lambda i: (0, 0)),
            pl.BlockSpec((Hh, Dg), lambda i: (0, 0)),
            pl.BlockSpec((1, Dg), lambda i: (0, 0)),
            pl.BlockSpec((128, Dg), lambda i: (0, 0)),
        ],
        out_specs=pl.BlockSpec((bn, Dg), lambda i: (i, 0)),
        out_shape=jax.ShapeDtypeStruct((M, Dg), jnp.float32))(
            gxp, ea, gx1, xrep, gxrep, tesh, w1a, w1b, w1c, b1, w2, b2, msh)


def _finish_a(s0, s1, c0, c1, x, bn=1000):
    M, Dg = s0.shape
    grid = M // bn

    def kern(s0r, s1r, c0r, c1r, xr, tr, str_):
        c = jnp.maximum(c0r[:, 0:1] + c1r[:, 0:1], 1.0)
        t = (s0r[...] + s1r[...]) / c + xr[...]
        tr[...] = t

        @pl.when(pl.program_id(0) == 0)
        def _():
            str_[...] = jnp.zeros_like(str_)

        str_[0:1, :] += jnp.sum(t, axis=0, keepdims=True)
        str_[1:2, :] += jnp.sum(t * t, axis=0, keepdims=True)

    return pl.pallas_call(
        kern, grid=(grid,),
        in_specs=[
            pl.BlockSpec((bn, Dg), lambda i: (i, 0)),
            pl.BlockSpec((bn, Dg), lambda i: (i, 0)),
            pl.BlockSpec((bn, 128), lambda i: (i, 0)),
            pl.BlockSpec((bn, 128), lambda i: (i, 0)),
            pl.BlockSpec((bn, Dg), lambda i: (i, 0)),
        ],
        out_specs=[pl.BlockSpec((bn, Dg), lambda i: (i, 0)),
                   pl.BlockSpec((8, Dg), lambda i: (0, 0))],
        out_shape=[jax.ShapeDtypeStruct((M, Dg), jnp.float32),
                   jax.ShapeDtypeStruct((8, Dg), jnp.float32)])(
            s0, s1, c0, c1, x)


def _finish_b(t, stats, g, b, count, bn=1000):
    M, Dg = t.shape
    grid = M // bn

    def kern(tr, sr, gr, br, outr):
        m = sr[0:1, :] / count
        v = sr[1:2, :] / count - m * m
        outr[...] = gr[...] * (tr[...] - m) / jnp.sqrt(v + 1e-5) + br[...]

    return pl.pallas_call(
        kern, grid=(grid,),
        in_specs=[
            pl.BlockSpec((bn, Dg), lambda i: (i, 0)),
            pl.BlockSpec((8, Dg), lambda i: (0, 0)),
            pl.BlockSpec((1, Dg), lambda i: (0, 0)),
            pl.BlockSpec((1, Dg), lambda i: (0, 0)),
        ],
        out_specs=pl.BlockSpec((bn, Dg), lambda i: (i, 0)),
        out_shape=jax.ShapeDtypeStruct((M, Dg), jnp.float32))(t, stats, g, b)


def _edge_mlp(ga, gb, bf, w1a, w1b, w1c, b1, w2, b2, w3, b3, lg, lb, bn=2048):
    M = bf.shape[0]
    grid = M // bn

    def kern(gar, gbr, bfr, w1ar, w1br, w1cr, b1r, w2r, b2r, w3r, b3r,
             lgr, lbr, outr):
        h = (_dot(gar[...], w1ar[...]) + _dot(gbr[...], w1br[...])
             + _dot(bfr[...], w1cr[...]) + b1r[...])
        h = jnp.maximum(h, 0.0)
        h = jnp.maximum(_dot(h, w2r[...]) + b2r[...], 0.0)
        h = _dot(h, w3r[...]) + b3r[...]
        r = bfr[...] + h
        m = jnp.mean(r, axis=-1, keepdims=True)
        v = jnp.mean((r - m) * (r - m), axis=-1, keepdims=True)
        outr[...] = lgr[...] * (r - m) / jnp.sqrt(v + 1e-5) + lbr[...]

    wspec = pl.BlockSpec((H, H), lambda i: (0, 0))
    bspec = pl.BlockSpec((1, H), lambda i: (0, 0))
    espec = pl.BlockSpec((bn, H), lambda i: (i, 0))
    return pl.pallas_call(
        kern, grid=(grid,),
        in_specs=[espec, espec, espec, wspec, wspec, wspec, bspec,
                  wspec, bspec, wspec, bspec, bspec, bspec],
        out_specs=espec,
        out_shape=jax.ShapeDtypeStruct((M, H), jnp.float32))(
            ga, gb, bf, w1a, w1b, w1c, b1, w2, b2, w3, b3, lg, lb)


def _prep_atoms(pos_pad):
    NP = pos_pad.shape[0]
    bn = 1024
    grid = NP // bn

    def kern(pr, outr):
        p = pr[...]
        ss = (p[:, 0:1] * p[:, 0:1] + p[:, 1:2] * p[:, 1:2]) \
            + p[:, 2:3] * p[:, 2:3]
        row = pl.program_id(0) * bn + lax.broadcasted_iota(
            jnp.int32, (bn, 1), 0).astype(jnp.float32)
        big = jnp.where(row >= float(N), 1e30, 0.0)
        z = jnp.zeros((bn, 12), jnp.float32)
        outr[...] = jnp.concatenate([p[:, 0:3], ss + big, z], axis=1)

    return pl.pallas_call(
        kern, grid=(grid,),
        in_specs=[pl.BlockSpec((bn, 16), lambda i: (i, 0))],
        out_specs=pl.BlockSpec((bn, 16), lambda i: (i, 0)),
        out_shape=jax.ShapeDtypeStruct((NP, 16), jnp.float32))(pos_pad)


def _prep_bonds(g0, g1):
    M = g0.shape[0]

    def kern(g0r, g1r, bpr, plr):
        bp = 0.5 * (g0r[...] + g1r[...])
        bsq = (bp[:, 0:1] * bp[:, 0:1] + bp[:, 1:2] * bp[:, 1:2]) \
            + bp[:, 2:3] * bp[:, 2:3]
        z1 = jnp.zeros((M, 1), jnp.float32)
        z = jnp.zeros((M, 11), jnp.float32)
        bpr[...] = jnp.concatenate([bp[:, 0:3], z1, bsq, z], axis=1)
        plr[...] = jnp.concatenate(
            [bp[:, 0:3], jnp.zeros((M, 13), jnp.float32)], axis=1)

    return pl.pallas_call(
        kern, grid=(1,),
        in_specs=[pl.BlockSpec((M, 16), lambda i: (0, 0)),
                  pl.BlockSpec((M, 16), lambda i: (0, 0))],
        out_specs=[pl.BlockSpec((M, 16), lambda i: (0, 0)),
                   pl.BlockSpec((M, 16), lambda i: (0, 0))],
        out_shape=[jax.ShapeDtypeStruct((M, 16), jnp.float32),
                   jax.ShapeDtypeStruct((M, 16), jnp.float32)])(g0, g1)


def _topk(bp_aug, a_aug, asq_row, rb=64):
    M = bp_aug.shape[0]
    NP = a_aug.shape[0]
    grid = M // rb

    def kern(bpr, ar, asqr, outr):
        bp = bpr[...]
        t = _dot_t(bp, ar[...])
        d2 = (bp[:, 4:5] + asqr[0:1, :]) - 2.0 * t
        iotaf = lax.broadcasted_iota(jnp.int32, (rb, NP), 1).astype(jnp.float32)
        cols = []
        v = d2
        for _ in range(K):
            m = jnp.min(v, axis=1, keepdims=True)
            cand = jnp.where(v <= m, iotaf, 3e7)
            idxf = jnp.min(cand, axis=1, keepdims=True)
            cols.append(idxf)
            v = jnp.where(iotaf == idxf, 1e30, v)
        outr[...] = jnp.concatenate(cols, axis=1).astype(jnp.int32)

    return pl.pallas_call(
        kern, grid=(grid,),
        in_specs=[pl.BlockSpec((rb, 16), lambda i: (i, 0)),
                  pl.BlockSpec((NP, 16), lambda i: (0, 0)),
                  pl.BlockSpec((8, NP), lambda i: (0, 0))],
        out_specs=pl.BlockSpec((rb, K), lambda i: (i, 0)),
        out_shape=jax.ShapeDtypeStruct((M, K), jnp.int32))(
            bp_aug, a_aug, asq_row)


def _sh5_cols(u):
    x = u[:, 0:1]
    y = u[:, 1:2]
    z = u[:, 2:3]
    return [SQ15 * x * y, SQ15 * y * z, SQ5 * 0.5 * (3.0 * z * z - 1.0),
            SQ15 * x * z, SQ15 * 0.5 * (x * x - y * y)]


def _tor_feat(gpos1, bposrep, grep1, prep2, mu, te1, teb1, te2, teb2, bn=1024):
    M = gpos1.shape[0]
    grid = M // bn
    sig = DMAX / NRBF

    def kern(g1r, bpr, gr1r, pr2r, mur, te1r, teb1r, te2r, teb2r,
             ear, teshr):
        ev = g1r[...] - bpr[...]
        d = jnp.sqrt(jnp.sum(ev * ev, axis=1, keepdims=True))
        rbf = jnp.exp(-(((d - mur[...]) / sig) ** 2))
        ea = _dot(jnp.maximum(_dot(rbf, te1r[...]) + teb1r[...], 0.0),
                  te2r[...]) + teb2r[...]
        ear[...] = ea
        u = ev / (d + 1e-8)
        sh5 = _sh5_cols(u)
        one = jnp.ones((bn, 1), jnp.float32)
        esh9 = [one, SQ3 * u[:, 0:1], SQ3 * u[:, 1:2], SQ3 * u[:, 2:3]] + sh5
        tbv = gr1r[...] - pr2r[...]
        db = jnp.sqrt(jnp.sum(tbv * tbv, axis=1, keepdims=True))
        ub = tbv / (db + 1e-8)
        tbsh = _sh5_cols(ub)
        cols = []
        for i in range(9):
            for j in range(5):
                cols.append(esh9[i] * tbsh[j])
        cols.append(jnp.zeros((bn, 128 - 45), jnp.float32))
        teshr[...] = jnp.concatenate(cols, axis=1)

    pspec = pl.BlockSpec((bn, 16), lambda i: (i, 0))
    wspec = pl.BlockSpec((H, H), lambda i: (0, 0))
    bspec = pl.BlockSpec((1, H), lambda i: (0, 0))
    return pl.pallas_call(
        kern, grid=(grid,),
        in_specs=[pspec, pspec, pspec, pspec, bspec, wspec, bspec, wspec,
                  bspec],
        out_specs=[pl.BlockSpec((bn, H), lambda i: (i, 0)),
                   pl.BlockSpec((bn, 128), lambda i: (i, 0))],
        out_shape=[jax.ShapeDtypeStruct((M, H), jnp.float32),
                   jax.ShapeDtypeStruct((M, 128), jnp.float32)])(
            gpos1, bposrep, grep1, prep2, mu, te1, teb1, te2, teb2)


def _tor_reduce(tp, smat, bn=1024, rb=64):
    M, Dg = tp.shape
    MB = M // K
    grid = M // bn

    def kern(tr, sr, outr, str_):
        r = _dot(sr[...], tr[...])
        row = pl.program_id(0) * rb + lax.broadcasted_iota(
            jnp.int32, (rb, 1), 0).astype(jnp.float32)
        msk = jnp.where(row < float(NROT), 1.0, 0.0)
        outr[...] = r

        @pl.when(pl.program_id(0) == 0)
        def _():
            str_[...] = jnp.zeros_like(str_)

        str_[0:1, :] += jnp.sum(r * msk, axis=0, keepdims=True)
        str_[1:2, :] += jnp.sum(r * r * msk, axis=0, keepdims=True)

    return pl.pallas_call(
        kern, grid=(grid,),
        in_specs=[pl.BlockSpec((bn, Dg), lambda i: (i, 0)),
                  pl.BlockSpec((rb, bn), lambda i: (0, 0))],
        out_specs=[pl.BlockSpec((rb, Dg), lambda i: (i, 0)),
                   pl.BlockSpec((8, Dg), lambda i: (0, 0))],
        out_shape=[jax.ShapeDtypeStruct((MB, Dg), jnp.float32),
                   jax.ShapeDtypeStruct((8, Dg), jnp.float32)])(tp, smat)


def _tu_mlp(tu_bn, f1, f2row, bn=1024):
    M, Dg = tu_bn.shape
    grid = M // bn

    def kern(tr, f1r, f2r, outr):
        h = jnp.maximum(_dot(tr[...], f1r[...]), 0.0)
        t = jnp.sum(h * f2r[...], axis=1, keepdims=True)
        t = jnp.tanh(t) * jnp.pi + 1e-4
        outr[...] = jnp.broadcast_to(t, (bn, 128))

    return pl.pallas_call(
        kern, grid=(grid,),
        in_specs=[pl.BlockSpec((bn, Dg), lambda i: (i, 0)),
                  pl.BlockSpec((Dg, 128), lambda i: (0, 0)),
                  pl.BlockSpec((1, 128), lambda i: (0, 0))],
        out_specs=pl.BlockSpec((bn, 128), lambda i: (i, 0)),
        out_shape=jax.ShapeDtypeStruct((M, 128), jnp.float32))(
            tu_bn, f1, f2row)


def _sin_axis(g0, g1, tu_b):
    M = g0.shape[0]

    def kern(g0r, g1r, tur, outr):
        av = g1r[...] - g0r[...]
        n = jnp.sqrt(jnp.sum(av * av, axis=1, keepdims=True))
        u = av / (n + 1e-8)
        outr[...] = jnp.sin(tur[:, 0:1]) * u

    return pl.pallas_call(
        kern, grid=(1,),
        in_specs=[pl.BlockSpec((M, 16), lambda i: (0, 0)),
                  pl.BlockSpec((M, 16), lambda i: (0, 0)),
                  pl.BlockSpec((M, 128), lambda i: (0, 0))],
        out_specs=pl.BlockSpec((M, 16), lambda i: (0, 0)),
        out_shape=jax.ShapeDtypeStruct((M, 16), jnp.float32))(g0, g1, tu_b)


def _new_pos(pos, gs, bn=1000):
    M = pos.shape[0]
    grid = M // bn

    def kern(pr, gr, outr):
        outr[...] = pr[...] + gr[:, 0:3]

    return pl.pallas_call(
        kern, grid=(grid,),
        in_specs=[pl.BlockSpec((bn, 3), lambda i: (i, 0)),
                  pl.BlockSpec((bn, 16), lambda i: (i, 0))],
        out_specs=pl.BlockSpec((bn, 3), lambda i: (i, 0)),
        out_shape=jax.ShapeDtypeStruct((M, 3), jnp.float32))(pos, gs)


def _pad_rows(a, rows, value=0):
    return jnp.pad(a, ((0, rows - a.shape[0]), (0, 0)), constant_values=value)


def _pad_1d(a, n, value):
    return jnp.pad(a, (0, n - a.shape[0]), constant_values=value)


def _row(v):
    return v.reshape(1, -1)


def kernel(atom_features, atom_pos, bond_features, bond_sh, bond_edge_index,
           spatial_features, spatial_sh, spatial_edge_index, rotatable_bonds,
           batch, update_instructs, params):
    p = params
    f32 = jnp.float32

    bd = _pad_1d(bond_edge_index[0], EP, 0)
    bs = _pad_1d(bond_edge_index[1], EP, 0)
    sd = _pad_1d(spatial_edge_index[0], EP, 0)
    bs_sc = _pad_1d(bond_edge_index[1], EP, N).reshape(EP // 128, 128)
    ss_sc = _pad_1d(spatial_edge_index[1], EP, N).reshape(EP // 128, 128)

    bf = _pad_rows(bond_features, EP)
    bshp = jnp.pad(bond_sh, ((0, EP - E), (0, 16 - SD)))
    sf = _pad_rows(spatial_features, EP)
    sshp = jnp.pad(spatial_sh, ((0, EP - E), (0, 16 - SD)))

    zeros128 = jnp.zeros((128, D), f32)
    ones128 = jnp.ones((EP, D), f32)

    cb = _sc_scatter_add(ones128, bs_sc, zeros128)
    cs = _sc_scatter_add(ones128, ss_sc, zeros128)
    cb0, cb1 = cb[0, :N], cb[1, :N]
    cs0, cs1 = cs[0, :N], cs[1, :N]

    x = atom_features

    def tconv(x, ef, esh, dstI, srcI2d, mx, w1, b1, w2, b2, msh, bg, bb,
              c0, c1):
        xp = _mm(x, mx)
        g = _sc_gather(xp, dstI)
        mshp = jnp.pad(msh, ((0, 16 - SD), (0, 0)))
        tp = _edge_dense(g, ef, esh, w1, _row(b1), w2, _row(b2), mshp)
        sums = _sc_scatter_add(tp, srcI2d, zeros128)
        t, stats = _finish_a(sums[0, :N], sums[1, :N], c0, c1, x)
        return _finish_b(t, stats, _row(bg), _row(bb), float(N))

    for l in range(5):
        x = tconv(x, bf, bshp, bd, bs_sc,
                  p['b_Mx'][l], p['b_fc1_W'][l], p['b_fc1_b'][l],
                  p['b_fc2_W'][l], p['b_fc2_b'][l], p['b_Msh'][l],
                  p['b_bn_g'][l], p['b_bn_b'][l], cb0, cb1)
        a = jnp.pad(_mm(x, p['eu_lin_W'][l]), ((0, 0), (0, D - H)))
        gab = _sc_gather(a, jnp.concatenate([bd, bs]))
        ga = gab[:EP, :H]
        gb = gab[EP:, :H]
        w1 = p['eu_fc1_W'][l]
        bf = _edge_mlp(ga, gb, bf, w1[:H], w1[H:2 * H], w1[2 * H:],
                       _row(p['eu_fc1_b'][l]), p['eu_fc2_W'][l],
                       _row(p['eu_fc2_b'][l]), p['eu_fc3_W'][l],
                       _row(p['eu_fc3_b'][l]), _row(p['eu_ln_g'][l]),
                       _row(p['eu_ln_b'][l]))
        x = tconv(x, sf, sshp, sd, ss_sc,
                  p['s_Mx'][l], p['s_fc1_W'][l], p['s_fc1_b'][l],
                  p['s_fc2_W'][l], p['s_fc2_b'][l], p['s_Msh'][l],
                  p['s_bn_g'][l], p['s_bn_b'][l], cs0, cs1)

    pos_tbl = jnp.pad(atom_pos, ((0, NUP - N), (0, 128 - 3)))

    beT = jnp.pad(bond_edge_index.T, ((0, 0), (0, 126)))
    rotP = _pad_1d(rotatable_bonds, NROTP, 0)
    grb = _sc_gather(beT, rotP, chunk=64)
    rb0 = grb[:, 0]
    rb1 = grb[:, 1]
    g0 = _sc_gather(pos_tbl, rb0, chunk=64)[:, :16]
    g1p = _sc_gather(pos_tbl, rb1, chunk=64)[:, :16]

    bp_aug, bp_plain = _prep_bonds(g0, g1p)
    a_aug = _prep_atoms(pos_tbl[:10240, :16])
    asq_row = jnp.pad(a_aug[:, 3:4].T, ((0, 7), (0, 0)))
    nn = _topk(bp_aug, a_aug, asq_row)
    ti1 = nn.reshape(-1)

    gpos1 = _sc_gather(pos_tbl, ti1)[:, :16]
    gx1 = _sc_gather(x, ti1)
    xp_t = _mm(x, p['t_Mx'])
    gxp = _sc_gather(xp_t, ti1)

    rep16 = lambda arr: jnp.repeat(arr, K, axis=0)
    bposrep = rep16(bp_plain)
    prep2 = jnp.repeat(pos_tbl[:ETP // 256, :16], 256, axis=0)
    grep1 = rep16(gpos1[:NROTP])
    xrep = jnp.repeat(x[:ETP // 256], 256, axis=0)
    gxrep = rep16(gx1[:NROTP])

    mu = _row(jnp.linspace(0.0, DMAX, NRBF).astype(f32))
    ea, tesh = _tor_feat(gpos1, bposrep, grep1, prep2, mu,
                         p['te_fc1_W'], _row(p['te_fc1_b']),
                         p['te_fc2_W'], _row(p['te_fc2_b']))

    tw1 = p['t_fc1_W']
    tmsh = jnp.pad(p['t_Msh'], ((0, 128 - SD * 5), (0, 0)))
    tp_t = _edge_dense_tor(gxp, ea, gx1, xrep, gxrep, tesh,
                           tw1[:H], tw1[H:H + NS], tw1[H + NS:],
                           _row(p['t_fc1_b']), p['t_fc2_W'],
                           _row(p['t_fc2_b']), tmsh)

    smat = (jnp.arange(1024) // K == jnp.arange(64)[:, None]).astype(f32) / K
    tu_pre, stats_t = _tor_reduce(tp_t, smat)
    tu_bn = _finish_b(tu_pre, stats_t, _row(p['t_bn_g']), _row(p['t_bn_b']),
                      float(NROT), bn=1024)
    f2row = _row(p['f2_W'][:, 0])
    tu_b = _tu_mlp(tu_bn, p['f1_W'], f2row)

    sv = jnp.pad(_sin_axis(g0, g1p, tu_b), ((0, 0), (0, 128 - 16)))
    uiP = _pad_1d(update_instructs, NUP, 0)
    gs = _sc_gather(sv, uiP)[:, :16]
    new_pos = _new_pos(atom_pos, gs[:N])

    return (new_pos, x, bf[:E], tu_b[:NROT, 0])


# speedup vs baseline: 1.0096x; 1.0096x over previous
import jax
import jax.numpy as jnp
from jax import lax
from jax.experimental import pallas as pl
from jax.experimental.pallas import tpu as pltpu
from jax.experimental.pallas import tpu_sc as plsc

N = 10000
D = 128
H = 32
E = 160000
SD = 9
NROT = 2000
K = 16
ET = NROT * K
NRBF = 32
DMAX = 5.0
NS = 128
SQ3 = 3.0 ** 0.5
SQ5 = 5.0 ** 0.5
SQ15 = 15.0 ** 0.5

NC = 2
NSUB = 16
NW = NC * NSUB

EP = 163840
ETP = 32768
NROTP = 2048
NUP = 12288
ACC = 10112


def _sc_mesh():
    return plsc.VectorSubcoreMesh(
        core_axis_name="c", subcore_axis_name="s",
        num_cores=NC, num_subcores=NSUB)


F0 = 0.31


def _core_split(nch_total):
    per_pair = nch_total // NSUB
    cpt0 = max(1, min(per_pair - 1, round(per_pair * F0)))
    return cpt0, per_pair - cpt0


def _sc_gather(table, idx, chunk=128):
    B = idx.shape[0]
    Dc = table.shape[1]
    dtype = table.dtype
    nch_total = B // chunk
    assert B % (NSUB * chunk) == 0
    cpt0, cpt1 = _core_split(nch_total)
    cptm = max(cpt0, cpt1)

    NB = max(1, min(4, cpt0, (131000 - cptm * chunk) // (chunk * Dc)))

    def body(table_hbm, idx_hbm, out_hbm, idx_v, *rest):
        bufs = rest[:NB]
        gsems = rest[NB:2 * NB]
        ssems = rest[2 * NB:3 * NB]
        c = lax.axis_index("c")
        s = lax.axis_index("s")

        def ring(cbase_fn, nchunks):
            base = cbase_fn(s) * chunk
            pltpu.sync_copy(idx_hbm.at[pl.ds(base, nchunks * chunk)],
                            idx_v.at[pl.ds(0, nchunks * chunk)])
            gd = {}
            sd = {}

            def start_gather(j):
                b = j % NB
                gd[j] = pltpu.async_copy(
                    table_hbm.at[idx_v.at[pl.ds(j * chunk, chunk)]],
                    bufs[b], gsems[b])

            for j in range(min(NB, nchunks)):
                start_gather(j)
            for j in range(nchunks):
                b = j % NB
                if j >= 1 and (j - 1 + NB) < nchunks:
                    sd[j - 1].wait()
                    start_gather(j - 1 + NB)
                gd[j].wait()
                sd[j] = pltpu.async_copy(
                    bufs[b], out_hbm.at[pl.ds(base + j * chunk, chunk)],
                    ssems[b])
            for t in range(max(0, nchunks - NB), nchunks):
                sd[t].wait()

        @pl.when(c == 0)
        def _():
            ring(lambda s_: s_ * cpt0, cpt0)

        @pl.when(c == 1)
        def _():
            ring(lambda s_: NSUB * cpt0 + s_ * cpt1, cpt1)

    fn = pl.kernel(
        body,
        out_type=jax.ShapeDtypeStruct((B, Dc), dtype),
        mesh=_sc_mesh(),
        scratch_types=(
            [pltpu.VMEM((cptm * chunk,), jnp.int32)]
            + [pltpu.VMEM((chunk, Dc), dtype)] * NB
            + [pltpu.SemaphoreType.DMA] * (2 * NB)
        ))
    return fn(table, idx)


def _sc_scatter_add(vals, idx2d, zeros_chunk):
    B, Dc = vals.shape
    chunk = 128
    nch_total = B // chunk
    zrows = ACC // NSUB
    assert zrows * NSUB == ACC
    per_pair = nch_total // NSUB
    if per_pair >= 16 and per_pair % 8 == 0:
        cpt0 = min(per_pair - 8, max(8, round(per_pair * F0 / 8) * 8))
    else:
        cpt0 = per_pair - per_pair // 2
    cpt1 = per_pair - cpt0
    cptm = max(cpt0, cpt1)

    NB = min(2, cpt0)

    def body(vals_hbm, idx_hbm, zeros_hbm, out_hbm, idx_v, acc, *rest):
        bufs = rest[:NB]
        lsems = rest[NB:2 * NB]
        asems = rest[2 * NB:3 * NB]
        c = lax.axis_index("c")
        s = lax.axis_index("s")

        pltpu.sync_copy(zeros_hbm, bufs[0])
        zbase = s * zrows
        off = 0
        while off < zrows:
            sz = min(chunk, zrows - off)
            pltpu.sync_copy(bufs[0].at[pl.ds(0, sz)],
                            acc.at[pl.ds(zbase + off, sz)])
            off += sz
        plsc.subcore_barrier()

        def ring(rowbase, nchunks):
            pltpu.sync_copy(idx_hbm.at[pl.ds(rowbase, nchunks)],
                            idx_v.at[pl.ds(0, nchunks)])
            vbase = rowbase * chunk
            ld = {}
            ad = {}

            def start_load(j):
                b = j % NB
                ld[j] = pltpu.async_copy(
                    vals_hbm.at[pl.ds(vbase + j * chunk, chunk)],
                    bufs[b], lsems[b])

            for j in range(min(NB, nchunks)):
                start_load(j)
            for j in range(nchunks):
                b = j % NB
                if j >= 1 and (j - 1 + NB) < nchunks:
                    ad[j - 1].wait()
                    start_load(j - 1 + NB)
                ld[j].wait()
                ad[j] = pltpu.async_copy(bufs[b], acc.at[idx_v.at[j]],
                                         asems[b], add=True)
            for t in range(max(0, nchunks - NB), nchunks):
                ad[t].wait()

        @pl.when(c == 0)
        def _():
            ring(s * cpt0, cpt0)

        @pl.when(c == 1)
        def _():
            ring(NSUB * cpt0 + s * cpt1, cpt1)

        plsc.subcore_barrier()

        off = 0
        while off < zrows:
            sz = min(chunk, zrows - off)
            pltpu.sync_copy(acc.at[pl.ds(zbase + off, sz)],
                            bufs[0].at[pl.ds(0, sz)])
            pltpu.sync_copy(bufs[0].at[pl.ds(0, sz)],
                            out_hbm.at[c, pl.ds(zbase + off, sz)])
            off += sz

    fn = pl.kernel(
        body,
        out_type=jax.ShapeDtypeStruct((NC, ACC, Dc), jnp.float32),
        mesh=_sc_mesh(),
        scratch_types=(
            [pltpu.VMEM((cptm, 128), jnp.int32),
             pltpu.VMEM_SHARED((ACC, Dc), jnp.float32)]
            + [pltpu.VMEM((chunk, Dc), jnp.float32)] * NB
            + [pltpu.SemaphoreType.DMA] * (2 * NB)
        ))
    return fn(vals, idx2d, zeros_chunk)


def _dot(a, b):
    return jax.lax.dot_general(a, b, (((1,), (0,)), ((), ())),
                               preferred_element_type=jnp.float32)


def _dot_t(a, b):
    return jax.lax.dot_general(a, b, (((1,), (1,)), ((), ())),
                               preferred_element_type=jnp.float32)


def _mm(x, w, bn=1000):
    M, Kc = x.shape
    Do = w.shape[1]
    grid = M // bn
    assert M % bn == 0

    def kern(xr, wr, o):
        o[...] = _dot(xr[...], wr[...])

    return pl.pallas_call(
        kern, grid=(grid,),
        in_specs=[pl.BlockSpec((bn, Kc), lambda i: (i, 0)),
                  pl.BlockSpec((Kc, Do), lambda i: (0, 0))],
        out_specs=pl.BlockSpec((bn, Do), lambda i: (i, 0)),
        out_shape=jax.ShapeDtypeStruct((M, Do), jnp.float32))(x, w)


def _edge_dense(g, ef, esh, w1, b1, w2, b2, msh, bn=1024):
    M, Dg = g.shape
    F = ef.shape[1]
    Se = esh.shape[1]
    Hh = w1.shape[1]
    grid = M // bn

    def kern(gr, efr, eshr, w1r, b1r, w2r, b2r, mshr, outr):
        h = jnp.maximum(_dot(efr[...], w1r[...]) + b1r[...], 0.0)
        w = _dot(h, w2r[...]) + b2r[...]
        outr[...] = gr[...] * w * _dot(eshr[...], mshr[...])

    return pl.pallas_call(
        kern, grid=(grid,),
        in_specs=[
            pl.BlockSpec((bn, Dg), lambda i: (i, 0)),
            pl.BlockSpec((bn, F), lambda i: (i, 0)),
            pl.BlockSpec((bn, Se), lambda i: (i, 0)),
            pl.BlockSpec((F, Hh), lambda i: (0, 0)),
            pl.BlockSpec((1, Hh), lambda i: (0, 0)),
            pl.BlockSpec((Hh, Dg), lambda i: (0, 0)),
            pl.BlockSpec((1, Dg), lambda i: (0, 0)),
            pl.BlockSpec((Se, Dg), lambda i: (0, 0)),
        ],
        out_specs=pl.BlockSpec((bn, Dg), lambda i: (i, 0)),
        out_shape=jax.ShapeDtypeStruct((M, Dg), jnp.float32))(
            g, ef, esh, w1, b1, w2, b2, msh)


def _edge_dense_tor(gxp, ea, gx1, xrep, gxrep, tesh,
                    w1a, w1b, w1c, b1, w2, b2, msh, bn=512):
    M, Dg = gxp.shape
    Hh = w1a.shape[1]
    grid = M // bn

    def kern(gr, ear, g1r, xrr, gxrr, teshr,
             w1ar, w1br, w1cr, b1r, w2r, b2r, mshr, outr):
        h = (_dot(ear[...], w1ar[...]) + _dot(g1r[...], w1br[...])
             + _dot(xrr[...] + gxrr[...], w1cr[...]) + b1r[...])
        h = jnp.maximum(h, 0.0)
        w = _dot(h, w2r[...]) + b2r[...]
        outr[...] = gr[...] * w * _dot(teshr[...], mshr[...])

    return pl.pallas_call(
        kern, grid=(grid,),
        in_specs=[
            pl.BlockSpec((bn, Dg), lambda i: (i, 0)),
            pl.BlockSpec((bn, H), lambda i: (i, 0)),
            pl.BlockSpec((bn, D), lambda i: (i, 0)),
            pl.BlockSpec((bn, D), lambda i: (i, 0)),
            pl.BlockSpec((bn, D), lambda i: (i, 0)),
            pl.BlockSpec((bn, 128), lambda i: (i, 0)),
            pl.BlockSpec((H, Hh), lambda i: (0, 0)),
            pl.BlockSpec((D, Hh), lambda i: (0, 0)),
            pl.BlockSpec((D, Hh), lambda i: (0, 0)),
            pl.BlockSpec((1, Hh), lambda i: (0, 0)),
            pl.BlockSpec((Hh, Dg), lambda i: (0, 0)),
            pl.BlockSpec((1, Dg), lambda i: (0, 0)),
            pl.BlockSpec((128, Dg), lambda i: (0, 0)),
        ],
        out_specs=pl.BlockSpec((bn, Dg), lambda i: (i, 0)),
        out_shape=jax.ShapeDtypeStruct((M, Dg), jnp.float32))(
            gxp, ea, gx1, xrep, gxrep, tesh, w1a, w1b, w1c, b1, w2, b2, msh)


def _finish_a(s0, s1, c0, c1, x, bn=1000):
    M, Dg = s0.shape
    grid = M // bn

    def kern(s0r, s1r, c0r, c1r, xr, tr, str_):
        c = jnp.maximum(c0r[:, 0:1] + c1r[:, 0:1], 1.0)
        t = (s0r[...] + s1r[...]) / c + xr[...]
        tr[...] = t

        @pl.when(pl.program_id(0) == 0)
        def _():
            str_[...] = jnp.zeros_like(str_)

        str_[0:1, :] += jnp.sum(t, axis=0, keepdims=True)
        str_[1:2, :] += jnp.sum(t * t, axis=0, keepdims=True)

    return pl.pallas_call(
        kern, grid=(grid,),
        in_specs=[
            pl.BlockSpec((bn, Dg), lambda i: (i, 0)),
            pl.BlockSpec((bn, Dg), lambda i: (i, 0)),
            pl.BlockSpec((bn, 128), lambda i: (i, 0)),
            pl.BlockSpec((bn, 128), lambda i: (i, 0)),
            pl.BlockSpec((bn, Dg), lambda i: (i, 0)),
        ],
        out_specs=[pl.BlockSpec((bn, Dg), lambda i: (i, 0)),
                   pl.BlockSpec((8, Dg), lambda i: (0, 0))],
        out_shape=[jax.ShapeDtypeStruct((M, Dg), jnp.float32),
                   jax.ShapeDtypeStruct((8, Dg), jnp.float32)])(
            s0, s1, c0, c1, x)


def _finish_b(t, stats, g, b, count, bn=1000):
    M, Dg = t.shape
    grid = M // bn

    def kern(tr, sr, gr, br, outr):
        m = sr[0:1, :] / count
        v = sr[1:2, :] / count - m * m
        outr[...] = gr[...] * (tr[...] - m) / jnp.sqrt(v + 1e-5) + br[...]

    return pl.pallas_call(
        kern, grid=(grid,),
        in_specs=[
            pl.BlockSpec((bn, Dg), lambda i: (i, 0)),
            pl.BlockSpec((8, Dg), lambda i: (0, 0)),
            pl.BlockSpec((1, Dg), lambda i: (0, 0)),
            pl.BlockSpec((1, Dg), lambda i: (0, 0)),
        ],
        out_specs=pl.BlockSpec((bn, Dg), lambda i: (i, 0)),
        out_shape=jax.ShapeDtypeStruct((M, Dg), jnp.float32))(t, stats, g, b)


def _edge_mlp(ga, gb, bf, w1a, w1b, w1c, b1, w2, b2, w3, b3, lg, lb, bn=2048):
    M = bf.shape[0]
    grid = M // bn

    def kern(gar, gbr, bfr, w1ar, w1br, w1cr, b1r, w2r, b2r, w3r, b3r,
             lgr, lbr, outr):
        h = (_dot(gar[...], w1ar[...]) + _dot(gbr[...], w1br[...])
             + _dot(bfr[...], w1cr[...]) + b1r[...])
        h = jnp.maximum(h, 0.0)
        h = jnp.maximum(_dot(h, w2r[...]) + b2r[...], 0.0)
        h = _dot(h, w3r[...]) + b3r[...]
        r = bfr[...] + h
        m = jnp.mean(r, axis=-1, keepdims=True)
        v = jnp.mean((r - m) * (r - m), axis=-1, keepdims=True)
        outr[...] = lgr[...] * (r - m) / jnp.sqrt(v + 1e-5) + lbr[...]

    wspec = pl.BlockSpec((H, H), lambda i: (0, 0))
    bspec = pl.BlockSpec((1, H), lambda i: (0, 0))
    espec = pl.BlockSpec((bn, H), lambda i: (i, 0))
    return pl.pallas_call(
        kern, grid=(grid,),
        in_specs=[espec, espec, espec, wspec, wspec, wspec, bspec,
                  wspec, bspec, wspec, bspec, bspec, bspec],
        out_specs=espec,
        out_shape=jax.ShapeDtypeStruct((M, H), jnp.float32))(
            ga, gb, bf, w1a, w1b, w1c, b1, w2, b2, w3, b3, lg, lb)


def _prep_atoms(pos_pad):
    NP = pos_pad.shape[0]
    bn = 1024
    grid = NP // bn

    def kern(pr, outr):
        p = pr[...]
        ss = (p[:, 0:1] * p[:, 0:1] + p[:, 1:2] * p[:, 1:2]) \
            + p[:, 2:3] * p[:, 2:3]
        row = pl.program_id(0) * bn + lax.broadcasted_iota(
            jnp.int32, (bn, 1), 0).astype(jnp.float32)
        big = jnp.where(row >= float(N), 1e30, 0.0)
        z = jnp.zeros((bn, 12), jnp.float32)
        outr[...] = jnp.concatenate([p[:, 0:3], ss + big, z], axis=1)

    return pl.pallas_call(
        kern, grid=(grid,),
        in_specs=[pl.BlockSpec((bn, 16), lambda i: (i, 0))],
        out_specs=pl.BlockSpec((bn, 16), lambda i: (i, 0)),
        out_shape=jax.ShapeDtypeStruct((NP, 16), jnp.float32))(pos_pad)


def _prep_bonds(g0, g1):
    M = g0.shape[0]

    def kern(g0r, g1r, bpr, plr):
        bp = 0.5 * (g0r[...] + g1r[...])
        bsq = (bp[:, 0:1] * bp[:, 0:1] + bp[:, 1:2] * bp[:, 1:2]) \
            + bp[:, 2:3] * bp[:, 2:3]
        z1 = jnp.zeros((M, 1), jnp.float32)
        z = jnp.zeros((M, 11), jnp.float32)
        bpr[...] = jnp.concatenate([bp[:, 0:3], z1, bsq, z], axis=1)
        plr[...] = jnp.concatenate(
            [bp[:, 0:3], jnp.zeros((M, 13), jnp.float32)], axis=1)

    return pl.pallas_call(
        kern, grid=(1,),
        in_specs=[pl.BlockSpec((M, 16), lambda i: (0, 0)),
                  pl.BlockSpec((M, 16), lambda i: (0, 0))],
        out_specs=[pl.BlockSpec((M, 16), lambda i: (0, 0)),
                   pl.BlockSpec((M, 16), lambda i: (0, 0))],
        out_shape=[jax.ShapeDtypeStruct((M, 16), jnp.float32),
                   jax.ShapeDtypeStruct((M, 16), jnp.float32)])(g0, g1)


def _topk(bp_aug, a_aug, asq_row, rb=64):
    M = bp_aug.shape[0]
    NP = a_aug.shape[0]
    grid = M // rb

    def kern(bpr, ar, asqr, outr):
        bp = bpr[...]
        t = _dot_t(bp, ar[...])
        d2 = (bp[:, 4:5] + asqr[0:1, :]) - 2.0 * t
        iotaf = lax.broadcasted_iota(jnp.int32, (rb, NP), 1).astype(jnp.float32)
        cols = []
        v = d2
        for _ in range(K):
            m = jnp.min(v, axis=1, keepdims=True)
            cand = jnp.where(v <= m, iotaf, 3e7)
            idxf = jnp.min(cand, axis=1, keepdims=True)
            cols.append(idxf)
            v = jnp.where(iotaf == idxf, 1e30, v)
        outr[...] = jnp.concatenate(cols, axis=1).astype(jnp.int32)

    return pl.pallas_call(
        kern, grid=(grid,),
        in_specs=[pl.BlockSpec((rb, 16), lambda i: (i, 0)),
                  pl.BlockSpec((NP, 16), lambda i: (0, 0)),
                  pl.BlockSpec((8, NP), lambda i: (0, 0))],
        out_specs=pl.BlockSpec((rb, K), lambda i: (i, 0)),
        out_shape=jax.ShapeDtypeStruct((M, K), jnp.int32))(
            bp_aug, a_aug, asq_row)


def _sh5_cols(u):
    x = u[:, 0:1]
    y = u[:, 1:2]
    z = u[:, 2:3]
    return [SQ15 * x * y, SQ15 * y * z, SQ5 * 0.5 * (3.0 * z * z - 1.0),
            SQ15 * x * z, SQ15 * 0.5 * (x * x - y * y)]


def _tor_feat(gpos1, bposrep, grep1, prep2, mu, te1, teb1, te2, teb2, bn=1024):
    M = gpos1.shape[0]
    grid = M // bn
    sig = DMAX / NRBF

    def kern(g1r, bpr, gr1r, pr2r, mur, te1r, teb1r, te2r, teb2r,
             ear, teshr):
        ev = g1r[...] - bpr[...]
        d = jnp.sqrt(jnp.sum(ev * ev, axis=1, keepdims=True))
        rbf = jnp.exp(-(((d - mur[...]) / sig) ** 2))
        ea = _dot(jnp.maximum(_dot(rbf, te1r[...]) + teb1r[...], 0.0),
                  te2r[...]) + teb2r[...]
        ear[...] = ea
        u = ev / (d + 1e-8)
        sh5 = _sh5_cols(u)
        one = jnp.ones((bn, 1), jnp.float32)
        esh9 = [one, SQ3 * u[:, 0:1], SQ3 * u[:, 1:2], SQ3 * u[:, 2:3]] + sh5
        tbv = gr1r[...] - pr2r[...]
        db = jnp.sqrt(jnp.sum(tbv * tbv, axis=1, keepdims=True))
        ub = tbv / (db + 1e-8)
        tbsh = _sh5_cols(ub)
        cols = []
        for i in range(9):
            for j in range(5):
                cols.append(esh9[i] * tbsh[j])
        cols.append(jnp.zeros((bn, 128 - 45), jnp.float32))
        teshr[...] = jnp.concatenate(cols, axis=1)

    pspec = pl.BlockSpec((bn, 16), lambda i: (i, 0))
    wspec = pl.BlockSpec((H, H), lambda i: (0, 0))
    bspec = pl.BlockSpec((1, H), lambda i: (0, 0))
    return pl.pallas_call(
        kern, grid=(grid,),
        in_specs=[pspec, pspec, pspec, pspec, bspec, wspec, bspec, wspec,
                  bspec],
        out_specs=[pl.BlockSpec((bn, H), lambda i: (i, 0)),
                   pl.BlockSpec((bn, 128), lambda i: (i, 0))],
        out_shape=[jax.ShapeDtypeStruct((M, H), jnp.float32),
                   jax.ShapeDtypeStruct((M, 128), jnp.float32)])(
            gpos1, bposrep, grep1, prep2, mu, te1, teb1, te2, teb2)


def _tor_reduce(tp, smat, bn=1024, rb=64):
    M, Dg = tp.shape
    MB = M // K
    grid = M // bn

    def kern(tr, sr, outr, str_):
        r = _dot(sr[...], tr[...])
        row = pl.program_id(0) * rb + lax.broadcasted_iota(
            jnp.int32, (rb, 1), 0).astype(jnp.float32)
        msk = jnp.where(row < float(NROT), 1.0, 0.0)
        outr[...] = r

        @pl.when(pl.program_id(0) == 0)
        def _():
            str_[...] = jnp.zeros_like(str_)

        str_[0:1, :] += jnp.sum(r * msk, axis=0, keepdims=True)
        str_[1:2, :] += jnp.sum(r * r * msk, axis=0, keepdims=True)

    return pl.pallas_call(
        kern, grid=(grid,),
        in_specs=[pl.BlockSpec((bn, Dg), lambda i: (i, 0)),
                  pl.BlockSpec((rb, bn), lambda i: (0, 0))],
        out_specs=[pl.BlockSpec((rb, Dg), lambda i: (i, 0)),
                   pl.BlockSpec((8, Dg), lambda i: (0, 0))],
        out_shape=[jax.ShapeDtypeStruct((MB, Dg), jnp.float32),
                   jax.ShapeDtypeStruct((8, Dg), jnp.float32)])(tp, smat)


def _tu_mlp(tu_bn, f1, f2row, bn=1024):
    M, Dg = tu_bn.shape
    grid = M // bn

    def kern(tr, f1r, f2r, outr):
        h = jnp.maximum(_dot(tr[...], f1r[...]), 0.0)
        t = jnp.sum(h * f2r[...], axis=1, keepdims=True)
        t = jnp.tanh(t) * jnp.pi + 1e-4
        outr[...] = jnp.broadcast_to(t, (bn, 128))

    return pl.pallas_call(
        kern, grid=(grid,),
        in_specs=[pl.BlockSpec((bn, Dg), lambda i: (i, 0)),
                  pl.BlockSpec((Dg, 128), lambda i: (0, 0)),
                  pl.BlockSpec((1, 128), lambda i: (0, 0))],
        out_specs=pl.BlockSpec((bn, 128), lambda i: (i, 0)),
        out_shape=jax.ShapeDtypeStruct((M, 128), jnp.float32))(
            tu_bn, f1, f2row)


def _sin_axis(g0, g1, tu_b):
    M = g0.shape[0]

    def kern(g0r, g1r, tur, outr):
        av = g1r[...] - g0r[...]
        n = jnp.sqrt(jnp.sum(av * av, axis=1, keepdims=True))
        u = av / (n + 1e-8)
        outr[...] = jnp.sin(tur[:, 0:1]) * u

    return pl.pallas_call(
        kern, grid=(1,),
        in_specs=[pl.BlockSpec((M, 16), lambda i: (0, 0)),
                  pl.BlockSpec((M, 16), lambda i: (0, 0)),
                  pl.BlockSpec((M, 128), lambda i: (0, 0))],
        out_specs=pl.BlockSpec((M, 16), lambda i: (0, 0)),
        out_shape=jax.ShapeDtypeStruct((M, 16), jnp.float32))(g0, g1, tu_b)


def _new_pos(pos, gs, bn=1000):
    M = pos.shape[0]
    grid = M // bn

    def kern(pr, gr, outr):
        outr[...] = pr[...] + gr[:, 0:3]

    return pl.pallas_call(
        kern, grid=(grid,),
        in_specs=[pl.BlockSpec((bn, 3), lambda i: (i, 0)),
                  pl.BlockSpec((bn, 16), lambda i: (i, 0))],
        out_specs=pl.BlockSpec((bn, 3), lambda i: (i, 0)),
        out_shape=jax.ShapeDtypeStruct((M, 3), jnp.float32))(pos, gs)


def _pad_rows(a, rows, value=0):
    return jnp.pad(a, ((0, rows - a.shape[0]), (0, 0)), constant_values=value)


def _pad_1d(a, n, value):
    return jnp.pad(a, (0, n - a.shape[0]), constant_values=value)


def _row(v):
    return v.reshape(1, -1)


def kernel(atom_features, atom_pos, bond_features, bond_sh, bond_edge_index,
           spatial_features, spatial_sh, spatial_edge_index, rotatable_bonds,
           batch, update_instructs, params):
    p = params
    f32 = jnp.float32

    bd = _pad_1d(bond_edge_index[0], EP, 0)
    bs = _pad_1d(bond_edge_index[1], EP, 0)
    sd = _pad_1d(spatial_edge_index[0], EP, 0)
    bs_sc = _pad_1d(bond_edge_index[1], EP, N).reshape(EP // 128, 128)
    ss_sc = _pad_1d(spatial_edge_index[1], EP, N).reshape(EP // 128, 128)

    bf = _pad_rows(bond_features, EP)
    bshp = jnp.pad(bond_sh, ((0, EP - E), (0, 16 - SD)))
    sf = _pad_rows(spatial_features, EP)
    sshp = jnp.pad(spatial_sh, ((0, EP - E), (0, 16 - SD)))

    zeros128 = jnp.zeros((128, D), f32)
    ones128 = jnp.ones((EP, D), f32)

    cb = _sc_scatter_add(ones128, bs_sc, zeros128)
    cs = _sc_scatter_add(ones128, ss_sc, zeros128)
    cb0, cb1 = cb[0, :N], cb[1, :N]
    cs0, cs1 = cs[0, :N], cs[1, :N]

    x = atom_features

    def tconv(x, ef, esh, dstI, srcI2d, mx, w1, b1, w2, b2, msh, bg, bb,
              c0, c1):
        xp = _mm(x, mx)
        g = _sc_gather(xp, dstI)
        mshp = jnp.pad(msh, ((0, 16 - SD), (0, 0)))
        tp = _edge_dense(g, ef, esh, w1, _row(b1), w2, _row(b2), mshp)
        sums = _sc_scatter_add(tp, srcI2d, zeros128)
        t, stats = _finish_a(sums[0, :N], sums[1, :N], c0, c1, x)
        return _finish_b(t, stats, _row(bg), _row(bb), float(N))

    for l in range(5):
        x = tconv(x, bf, bshp, bd, bs_sc,
                  p['b_Mx'][l], p['b_fc1_W'][l], p['b_fc1_b'][l],
                  p['b_fc2_W'][l], p['b_fc2_b'][l], p['b_Msh'][l],
                  p['b_bn_g'][l], p['b_bn_b'][l], cb0, cb1)
        a = jnp.pad(_mm(x, p['eu_lin_W'][l]), ((0, 0), (0, D - H)))
        gab = _sc_gather(a, jnp.concatenate([bd, bs]))
        ga = gab[:EP, :H]
        gb = gab[EP:, :H]
        w1 = p['eu_fc1_W'][l]
        bf = _edge_mlp(ga, gb, bf, w1[:H], w1[H:2 * H], w1[2 * H:],
                       _row(p['eu_fc1_b'][l]), p['eu_fc2_W'][l],
                       _row(p['eu_fc2_b'][l]), p['eu_fc3_W'][l],
                       _row(p['eu_fc3_b'][l]), _row(p['eu_ln_g'][l]),
                       _row(p['eu_ln_b'][l]))
        x = tconv(x, sf, sshp, sd, ss_sc,
                  p['s_Mx'][l], p['s_fc1_W'][l], p['s_fc1_b'][l],
                  p['s_fc2_W'][l], p['s_fc2_b'][l], p['s_Msh'][l],
                  p['s_bn_g'][l], p['s_bn_b'][l], cs0, cs1)

    pos_tbl = jnp.pad(atom_pos, ((0, NUP - N), (0, 128 - 3)))

    beT = jnp.pad(bond_edge_index.T, ((0, 0), (0, 126)))
    rotP = _pad_1d(rotatable_bonds, NROTP, 0)
    grb = _sc_gather(beT, rotP, chunk=64)
    rb0 = grb[:, 0]
    rb1 = grb[:, 1]
    g0 = _sc_gather(pos_tbl, rb0, chunk=64)[:, :16]
    g1p = _sc_gather(pos_tbl, rb1, chunk=64)[:, :16]

    bp_aug, bp_plain = _prep_bonds(g0, g1p)
    a_aug = _prep_atoms(pos_tbl[:10240, :16])
    asq_row = jnp.pad(a_aug[:, 3:4].T, ((0, 7), (0, 0)))
    nn = _topk(bp_aug, a_aug, asq_row)
    ti1 = nn.reshape(-1)

    gpos1 = _sc_gather(pos_tbl, ti1)[:, :16]
    gx1 = _sc_gather(x, ti1)
    xp_t = _mm(x, p['t_Mx'])
    gxp = _sc_gather(xp_t, ti1)

    rep16 = lambda arr: jnp.repeat(arr, K, axis=0)
    bposrep = rep16(bp_plain)
    prep2 = jnp.repeat(pos_tbl[:ETP // 256, :16], 256, axis=0)
    grep1 = rep16(gpos1[:NROTP])
    xrep = jnp.repeat(x[:ETP // 256], 256, axis=0)
    gxrep = rep16(gx1[:NROTP])

    mu = _row(jnp.linspace(0.0, DMAX, NRBF).astype(f32))
    ea, tesh = _tor_feat(gpos1, bposrep, grep1, prep2, mu,
                         p['te_fc1_W'], _row(p['te_fc1_b']),
                         p['te_fc2_W'], _row(p['te_fc2_b']))

    tw1 = p['t_fc1_W']
    tmsh = jnp.pad(p['t_Msh'], ((0, 128 - SD * 5), (0, 0)))
    tp_t = _edge_dense_tor(gxp, ea, gx1, xrep, gxrep, tesh,
                           tw1[:H], tw1[H:H + NS], tw1[H + NS:],
                           _row(p['t_fc1_b']), p['t_fc2_W'],
                           _row(p['t_fc2_b']), tmsh)

    smat = (jnp.arange(1024) // K == jnp.arange(64)[:, None]).astype(f32) / K
    tu_pre, stats_t = _tor_reduce(tp_t, smat)
    tu_bn = _finish_b(tu_pre, stats_t, _row(p['t_bn_g']), _row(p['t_bn_b']),
                      float(NROT), bn=1024)
    f2row = _row(p['f2_W'][:, 0])
    tu_b = _tu_mlp(tu_bn, p['f1_W'], f2row)

    sv = jnp.pad(_sin_axis(g0, g1p, tu_b), ((0, 0), (0, 128 - 16)))
    uiP = _pad_1d(update_instructs, NUP, 0)
    gs = _sc_gather(sv, uiP)[:, :16]
    new_pos = _new_pos(atom_pos, gs[:N])

    return (new_pos, x, bf[:E], tu_b[:NROT, 0])


# speedup vs baseline: 1.0528x; 1.0428x over previous
import jax
import jax.numpy as jnp
from jax import lax
from jax.experimental import pallas as pl
from jax.experimental.pallas import tpu as pltpu
from jax.experimental.pallas import tpu_sc as plsc

N = 10000
D = 128
H = 32
E = 160000
SD = 9
NROT = 2000
K = 16
ET = NROT * K
NRBF = 32
DMAX = 5.0
NS = 128
SQ3 = 3.0 ** 0.5
SQ5 = 5.0 ** 0.5
SQ15 = 15.0 ** 0.5

NC = 2
NSUB = 16
NW = NC * NSUB

EP = 163840
ETP = 32768
NROTP = 2048
NUP = 12288
ACC = 10112


def _sc_mesh():
    return plsc.VectorSubcoreMesh(
        core_axis_name="c", subcore_axis_name="s",
        num_cores=NC, num_subcores=NSUB)


F0 = 0.5


def _core_split(nch_total):
    per_pair = nch_total // NSUB
    cpt0 = max(1, min(per_pair - 1, round(per_pair * F0)))
    return cpt0, per_pair - cpt0


def _sc_gather(table, idx, chunk=128):
    B = idx.shape[0]
    Dc = table.shape[1]
    dtype = table.dtype
    nch_total = B // chunk
    assert B % (NSUB * chunk) == 0
    cpt0, cpt1 = _core_split(nch_total)
    cptm = max(cpt0, cpt1)

    NB = max(1, min(4, cpt0, (131000 - cptm * chunk) // (chunk * Dc)))

    def body(table_hbm, idx_hbm, out_hbm, idx_v, *rest):
        bufs = rest[:NB]
        gsems = rest[NB:2 * NB]
        ssems = rest[2 * NB:3 * NB]
        c = lax.axis_index("c")
        s = lax.axis_index("s")

        def ring(cbase_fn, nchunks):
            base = cbase_fn(s) * chunk
            pltpu.sync_copy(idx_hbm.at[pl.ds(base, nchunks * chunk)],
                            idx_v.at[pl.ds(0, nchunks * chunk)])
            gd = {}
            sd = {}

            def start_gather(j):
                b = j % NB
                gd[j] = pltpu.async_copy(
                    table_hbm.at[idx_v.at[pl.ds(j * chunk, chunk)]],
                    bufs[b], gsems[b])

            for j in range(min(NB, nchunks)):
                start_gather(j)
            for j in range(nchunks):
                b = j % NB
                if j >= 1 and (j - 1 + NB) < nchunks:
                    sd[j - 1].wait()
                    start_gather(j - 1 + NB)
                gd[j].wait()
                sd[j] = pltpu.async_copy(
                    bufs[b], out_hbm.at[pl.ds(base + j * chunk, chunk)],
                    ssems[b])
            for t in range(max(0, nchunks - NB), nchunks):
                sd[t].wait()

        @pl.when(c == 0)
        def _():
            ring(lambda s_: s_ * cpt0, cpt0)

        @pl.when(c == 1)
        def _():
            ring(lambda s_: NSUB * cpt0 + s_ * cpt1, cpt1)

    fn = pl.kernel(
        body,
        out_type=jax.ShapeDtypeStruct((B, Dc), dtype),
        mesh=_sc_mesh(),
        scratch_types=(
            [pltpu.VMEM((cptm * chunk,), jnp.int32)]
            + [pltpu.VMEM((chunk, Dc), dtype)] * NB
            + [pltpu.SemaphoreType.DMA] * (2 * NB)
        ))
    return fn(table, idx)


def _sc_scatter_add(vals, idx2d, zeros_chunk):
    B, Dc = vals.shape
    chunk = 128
    nch_total = B // chunk
    zrows = ACC // NSUB
    assert zrows * NSUB == ACC
    per_pair = nch_total // NSUB
    if per_pair >= 16 and per_pair % 8 == 0:
        cpt0 = min(per_pair - 8, max(8, round(per_pair * F0 / 8) * 8))
    else:
        cpt0 = per_pair - per_pair // 2
    cpt1 = per_pair - cpt0
    cptm = max(cpt0, cpt1)

    NB = min(2, cpt0)

    def body(vals_hbm, idx_hbm, zeros_hbm, out_hbm, idx_v, acc, *rest):
        bufs = rest[:NB]
        lsems = rest[NB:2 * NB]
        asems = rest[2 * NB:3 * NB]
        c = lax.axis_index("c")
        s = lax.axis_index("s")

        pltpu.sync_copy(zeros_hbm, bufs[0])
        zbase = s * zrows
        off = 0
        while off < zrows:
            sz = min(chunk, zrows - off)
            pltpu.sync_copy(bufs[0].at[pl.ds(0, sz)],
                            acc.at[pl.ds(zbase + off, sz)])
            off += sz
        plsc.subcore_barrier()

        def ring(rowbase, nchunks):
            pltpu.sync_copy(idx_hbm.at[pl.ds(rowbase, nchunks)],
                            idx_v.at[pl.ds(0, nchunks)])
            vbase = rowbase * chunk
            ld = {}
            ad = {}

            def start_load(j):
                b = j % NB
                ld[j] = pltpu.async_copy(
                    vals_hbm.at[pl.ds(vbase + j * chunk, chunk)],
                    bufs[b], lsems[b])

            for j in range(min(NB, nchunks)):
                start_load(j)
            for j in range(nchunks):
                b = j % NB
                if j >= 1 and (j - 1 + NB) < nchunks:
                    ad[j - 1].wait()
                    start_load(j - 1 + NB)
                ld[j].wait()
                ad[j] = pltpu.async_copy(bufs[b], acc.at[idx_v.at[j]],
                                         asems[b], add=True)
            for t in range(max(0, nchunks - NB), nchunks):
                ad[t].wait()

        @pl.when(c == 0)
        def _():
            ring(s * cpt0, cpt0)

        @pl.when(c == 1)
        def _():
            ring(NSUB * cpt0 + s * cpt1, cpt1)

        plsc.subcore_barrier()

        off = 0
        while off < zrows:
            sz = min(chunk, zrows - off)
            pltpu.sync_copy(acc.at[pl.ds(zbase + off, sz)],
                            bufs[0].at[pl.ds(0, sz)])
            pltpu.sync_copy(bufs[0].at[pl.ds(0, sz)],
                            out_hbm.at[c, pl.ds(zbase + off, sz)])
            off += sz

    fn = pl.kernel(
        body,
        out_type=jax.ShapeDtypeStruct((NC, ACC, Dc), jnp.float32),
        mesh=_sc_mesh(),
        scratch_types=(
            [pltpu.VMEM((cptm, 128), jnp.int32),
             pltpu.VMEM_SHARED((ACC, Dc), jnp.float32)]
            + [pltpu.VMEM((chunk, Dc), jnp.float32)] * NB
            + [pltpu.SemaphoreType.DMA] * (2 * NB)
        ))
    return fn(vals, idx2d, zeros_chunk)


def _dot(a, b):
    return jax.lax.dot_general(a, b, (((1,), (0,)), ((), ())),
                               preferred_element_type=jnp.float32)


def _dot_t(a, b):
    return jax.lax.dot_general(a, b, (((1,), (1,)), ((), ())),
                               preferred_element_type=jnp.float32)


def _mm(x, w, bn=1000):
    M, Kc = x.shape
    Do = w.shape[1]
    grid = M // bn
    assert M % bn == 0

    def kern(xr, wr, o):
        o[...] = _dot(xr[...], wr[...])

    return pl.pallas_call(
        kern, grid=(grid,),
        in_specs=[pl.BlockSpec((bn, Kc), lambda i: (i, 0)),
                  pl.BlockSpec((Kc, Do), lambda i: (0, 0))],
        out_specs=pl.BlockSpec((bn, Do), lambda i: (i, 0)),
        out_shape=jax.ShapeDtypeStruct((M, Do), jnp.float32))(x, w)


def _edge_dense(g, ef, esh, w1, b1, w2, b2, msh, bn=1024):
    M, Dg = g.shape
    F = ef.shape[1]
    Se = esh.shape[1]
    Hh = w1.shape[1]
    grid = M // bn

    def kern(gr, efr, eshr, w1r, b1r, w2r, b2r, mshr, outr):
        h = jnp.maximum(_dot(efr[...], w1r[...]) + b1r[...], 0.0)
        w = _dot(h, w2r[...]) + b2r[...]
        outr[...] = gr[...] * w * _dot(eshr[...], mshr[...])

    return pl.pallas_call(
        kern, grid=(grid,),
        in_specs=[
            pl.BlockSpec((bn, Dg), lambda i: (i, 0)),
            pl.BlockSpec((bn, F), lambda i: (i, 0)),
            pl.BlockSpec((bn, Se), lambda i: (i, 0)),
            pl.BlockSpec((F, Hh), lambda i: (0, 0)),
            pl.BlockSpec((1, Hh), lambda i: (0, 0)),
            pl.BlockSpec((Hh, Dg), lambda i: (0, 0)),
            pl.BlockSpec((1, Dg), lambda i: (0, 0)),
            pl.BlockSpec((Se, Dg), lambda i: (0, 0)),
        ],
        out_specs=pl.BlockSpec((bn, Dg), lambda i: (i, 0)),
        out_shape=jax.ShapeDtypeStruct((M, Dg), jnp.float32))(
            g, ef, esh, w1, b1, w2, b2, msh)


def _edge_dense_tor(gxp, ea, gx1, xsmall, tesh,
                    w1a, w1b, w1c, b1, w2, b2, msh, bn=512):
    M, Dg = gxp.shape
    Hh = w1a.shape[1]
    grid = M // bn
    nb16 = bn // K
    nb256 = bn // 256

    def kern(gr, ear, g1r, g16r, xsr, teshr,
             w1ar, w1br, w1cr, b1r, w2r, b2r, mshr, outr):
        offx = (pl.program_id(0) * nb256) % 8
        xrep = _dot(_rep_mat(bn, 8, 256, offx), xsr[...])
        gxrep = _dot(_rep_mat(bn, nb16, K), g16r[...])
        h = (_dot(ear[...], w1ar[...]) + _dot(g1r[...], w1br[...])
             + _dot(xrep + gxrep, w1cr[...]) + b1r[...])
        h = jnp.maximum(h, 0.0)
        w = _dot(h, w2r[...]) + b2r[...]
        outr[...] = gr[...] * w * _dot(teshr[...], mshr[...])

    return pl.pallas_call(
        kern, grid=(grid,),
        in_specs=[
            pl.BlockSpec((bn, Dg), lambda i: (i, 0)),
            pl.BlockSpec((bn, H), lambda i: (i, 0)),
            pl.BlockSpec((bn, D), lambda i: (i, 0)),
            pl.BlockSpec((nb16, D), lambda i: (i, 0)),
            pl.BlockSpec((8, D), lambda i: ((i * nb256) // 8, 0)),
            pl.BlockSpec((bn, 128), lambda i: (i, 0)),
            pl.BlockSpec((H, Hh), lambda i: (0, 0)),
            pl.BlockSpec((D, Hh), lambda i: (0, 0)),
            pl.BlockSpec((D, Hh), lambda i: (0, 0)),
            pl.BlockSpec((1, Hh), lambda i: (0, 0)),
            pl.BlockSpec((Hh, Dg), lambda i: (0, 0)),
            pl.BlockSpec((1, Dg), lambda i: (0, 0)),
            pl.BlockSpec((128, Dg), lambda i: (0, 0)),
        ],
        out_specs=pl.BlockSpec((bn, Dg), lambda i: (i, 0)),
        out_shape=jax.ShapeDtypeStruct((M, Dg), jnp.float32))(
            gxp, ea, gx1, gx1[:M // K], xsmall, tesh,
            w1a, w1b, w1c, b1, w2, b2, msh)


def _finish_a(s0, s1, c0, c1, x, bn=1000):
    M, Dg = s0.shape
    grid = M // bn

    def kern(s0r, s1r, c0r, c1r, xr, tr, str_):
        c = jnp.maximum(c0r[:, 0:1] + c1r[:, 0:1], 1.0)
        t = (s0r[...] + s1r[...]) / c + xr[...]
        tr[...] = t

        @pl.when(pl.program_id(0) == 0)
        def _():
            str_[...] = jnp.zeros_like(str_)

        str_[0:1, :] += jnp.sum(t, axis=0, keepdims=True)
        str_[1:2, :] += jnp.sum(t * t, axis=0, keepdims=True)

    return pl.pallas_call(
        kern, grid=(grid,),
        in_specs=[
            pl.BlockSpec((bn, Dg), lambda i: (i, 0)),
            pl.BlockSpec((bn, Dg), lambda i: (i, 0)),
            pl.BlockSpec((bn, 128), lambda i: (i, 0)),
            pl.BlockSpec((bn, 128), lambda i: (i, 0)),
            pl.BlockSpec((bn, Dg), lambda i: (i, 0)),
        ],
        out_specs=[pl.BlockSpec((bn, Dg), lambda i: (i, 0)),
                   pl.BlockSpec((8, Dg), lambda i: (0, 0))],
        out_shape=[jax.ShapeDtypeStruct((M, Dg), jnp.float32),
                   jax.ShapeDtypeStruct((8, Dg), jnp.float32)])(
            s0, s1, c0, c1, x)


def _finish_b(t, stats, g, b, count, bn=1000):
    M, Dg = t.shape
    grid = M // bn

    def kern(tr, sr, gr, br, outr):
        m = sr[0:1, :] / count
        v = sr[1:2, :] / count - m * m
        outr[...] = gr[...] * (tr[...] - m) / jnp.sqrt(v + 1e-5) + br[...]

    return pl.pallas_call(
        kern, grid=(grid,),
        in_specs=[
            pl.BlockSpec((bn, Dg), lambda i: (i, 0)),
            pl.BlockSpec((8, Dg), lambda i: (0, 0)),
            pl.BlockSpec((1, Dg), lambda i: (0, 0)),
            pl.BlockSpec((1, Dg), lambda i: (0, 0)),
        ],
        out_specs=pl.BlockSpec((bn, Dg), lambda i: (i, 0)),
        out_shape=jax.ShapeDtypeStruct((M, Dg), jnp.float32))(t, stats, g, b)


def _edge_mlp(ga, gb, bf, w1a, w1b, w1c, b1, w2, b2, w3, b3, lg, lb, bn=2048):
    M = bf.shape[0]
    grid = M // bn

    def kern(gar, gbr, bfr, w1ar, w1br, w1cr, b1r, w2r, b2r, w3r, b3r,
             lgr, lbr, outr):
        h = (_dot(gar[...], w1ar[...]) + _dot(gbr[...], w1br[...])
             + _dot(bfr[...], w1cr[...]) + b1r[...])
        h = jnp.maximum(h, 0.0)
        h = jnp.maximum(_dot(h, w2r[...]) + b2r[...], 0.0)
        h = _dot(h, w3r[...]) + b3r[...]
        r = bfr[...] + h
        m = jnp.mean(r, axis=-1, keepdims=True)
        v = jnp.mean((r - m) * (r - m), axis=-1, keepdims=True)
        outr[...] = lgr[...] * (r - m) / jnp.sqrt(v + 1e-5) + lbr[...]

    wspec = pl.BlockSpec((H, H), lambda i: (0, 0))
    bspec = pl.BlockSpec((1, H), lambda i: (0, 0))
    espec = pl.BlockSpec((bn, H), lambda i: (i, 0))
    return pl.pallas_call(
        kern, grid=(grid,),
        in_specs=[espec, espec, espec, wspec, wspec, wspec, bspec,
                  wspec, bspec, wspec, bspec, bspec, bspec],
        out_specs=espec,
        out_shape=jax.ShapeDtypeStruct((M, H), jnp.float32))(
            ga, gb, bf, w1a, w1b, w1c, b1, w2, b2, w3, b3, lg, lb)


def _prep_atoms(pos_pad):
    NP = pos_pad.shape[0]
    bn = 1024
    grid = NP // bn

    def kern(pr, outr):
        p = pr[...]
        ss = (p[:, 0:1] * p[:, 0:1] + p[:, 1:2] * p[:, 1:2]) \
            + p[:, 2:3] * p[:, 2:3]
        row = pl.program_id(0) * bn + lax.broadcasted_iota(
            jnp.int32, (bn, 1), 0).astype(jnp.float32)
        big = jnp.where(row >= float(N), 1e30, 0.0)
        z = jnp.zeros((bn, 12), jnp.float32)
        outr[...] = jnp.concatenate([p[:, 0:3], ss + big, z], axis=1)

    return pl.pallas_call(
        kern, grid=(grid,),
        in_specs=[pl.BlockSpec((bn, 16), lambda i: (i, 0))],
        out_specs=pl.BlockSpec((bn, 16), lambda i: (i, 0)),
        out_shape=jax.ShapeDtypeStruct((NP, 16), jnp.float32))(pos_pad)


def _prep_bonds(g0, g1):
    M = g0.shape[0]

    def kern(g0r, g1r, bpr, plr):
        bp = 0.5 * (g0r[...] + g1r[...])
        bsq = (bp[:, 0:1] * bp[:, 0:1] + bp[:, 1:2] * bp[:, 1:2]) \
            + bp[:, 2:3] * bp[:, 2:3]
        z1 = jnp.zeros((M, 1), jnp.float32)
        z = jnp.zeros((M, 11), jnp.float32)
        bpr[...] = jnp.concatenate([bp[:, 0:3], z1, bsq, z], axis=1)
        plr[...] = jnp.concatenate(
            [bp[:, 0:3], jnp.zeros((M, 13), jnp.float32)], axis=1)

    return pl.pallas_call(
        kern, grid=(1,),
        in_specs=[pl.BlockSpec((M, 16), lambda i: (0, 0)),
                  pl.BlockSpec((M, 16), lambda i: (0, 0))],
        out_specs=[pl.BlockSpec((M, 16), lambda i: (0, 0)),
                   pl.BlockSpec((M, 16), lambda i: (0, 0))],
        out_shape=[jax.ShapeDtypeStruct((M, 16), jnp.float32),
                   jax.ShapeDtypeStruct((M, 16), jnp.float32)])(g0, g1)


def _topk(bp_aug, a_aug, asq_row, rb=64):
    M = bp_aug.shape[0]
    NP = a_aug.shape[0]
    grid = M // rb

    def kern(bpr, ar, asqr, outr):
        bp = bpr[...]
        t = _dot_t(bp, ar[...])
        d2 = (bp[:, 4:5] + asqr[0:1, :]) - 2.0 * t
        iotaf = lax.broadcasted_iota(jnp.int32, (rb, NP), 1).astype(jnp.float32)
        cols = []
        v = d2
        for _ in range(K):
            m = jnp.min(v, axis=1, keepdims=True)
            cand = jnp.where(v <= m, iotaf, 3e7)
            idxf = jnp.min(cand, axis=1, keepdims=True)
            cols.append(idxf)
            v = jnp.where(iotaf == idxf, 1e30, v)
        outr[...] = jnp.concatenate(cols, axis=1).astype(jnp.int32)

    return pl.pallas_call(
        kern, grid=(grid,),
        in_specs=[pl.BlockSpec((rb, 16), lambda i: (i, 0)),
                  pl.BlockSpec((NP, 16), lambda i: (0, 0)),
                  pl.BlockSpec((8, NP), lambda i: (0, 0))],
        out_specs=pl.BlockSpec((rb, K), lambda i: (i, 0)),
        out_shape=jax.ShapeDtypeStruct((M, K), jnp.int32))(
            bp_aug, a_aug, asq_row)


def _sh5_cols(u):
    x = u[:, 0:1]
    y = u[:, 1:2]
    z = u[:, 2:3]
    return [SQ15 * x * y, SQ15 * y * z, SQ5 * 0.5 * (3.0 * z * z - 1.0),
            SQ15 * x * z, SQ15 * 0.5 * (x * x - y * y)]


def _rep_mat(rows, cols, period, offset=0):
    ri = lax.broadcasted_iota(jnp.int32, (rows, cols), 0) // period + offset
    ci = lax.broadcasted_iota(jnp.int32, (rows, cols), 1)
    return (ri == ci).astype(jnp.float32)


def _tor_feat(gpos1, bp_plain, pos16, mu, te1, teb1, te2, teb2, bn=1024):
    M = gpos1.shape[0]
    grid = M // bn
    nb16 = bn // K
    nb256 = bn // 256
    sig = DMAX / NRBF

    def kern(g1r, bpr, gp16r, p16r, mur, te1r, teb1r, te2r, teb2r,
             ear, teshr):
        r16 = _rep_mat(bn, nb16, K)
        off256 = (pl.program_id(0) * nb256) % 8
        r256 = _rep_mat(bn, 8, 256, off256)
        bposrep = _dot(r16, bpr[...])
        grep1 = _dot(r16, gp16r[...])
        prep2 = _dot(r256, p16r[...])
        ev = g1r[...] - bposrep
        d = jnp.sqrt(jnp.sum(ev * ev, axis=1, keepdims=True))
        rbf = jnp.exp(-(((d - mur[...]) / sig) ** 2))
        ea = _dot(jnp.maximum(_dot(rbf, te1r[...]) + teb1r[...], 0.0),
                  te2r[...]) + teb2r[...]
        ear[...] = ea
        u = ev / (d + 1e-8)
        sh5 = _sh5_cols(u)
        one = jnp.ones((bn, 1), jnp.float32)
        esh9 = [one, SQ3 * u[:, 0:1], SQ3 * u[:, 1:2], SQ3 * u[:, 2:3]] + sh5
        tbv = grep1 - prep2
        db = jnp.sqrt(jnp.sum(tbv * tbv, axis=1, keepdims=True))
        ub = tbv / (db + 1e-8)
        tbsh = _sh5_cols(ub)
        cols = []
        for i in range(9):
            for j in range(5):
                cols.append(esh9[i] * tbsh[j])
        cols.append(jnp.zeros((bn, 128 - 45), jnp.float32))
        teshr[...] = jnp.concatenate(cols, axis=1)

    wspec = pl.BlockSpec((H, H), lambda i: (0, 0))
    bspec = pl.BlockSpec((1, H), lambda i: (0, 0))
    return pl.pallas_call(
        kern, grid=(grid,),
        in_specs=[pl.BlockSpec((bn, 16), lambda i: (i, 0)),
                  pl.BlockSpec((nb16, 16), lambda i: (i, 0)),
                  pl.BlockSpec((nb16, 16), lambda i: (i, 0)),
                  pl.BlockSpec((8, 16), lambda i: ((i * nb256) // 8, 0)),
                  bspec, wspec, bspec, wspec, bspec],
        out_specs=[pl.BlockSpec((bn, H), lambda i: (i, 0)),
                   pl.BlockSpec((bn, 128), lambda i: (i, 0))],
        out_shape=[jax.ShapeDtypeStruct((M, H), jnp.float32),
                   jax.ShapeDtypeStruct((M, 128), jnp.float32)])(
            gpos1, bp_plain, gpos1[:M // K], pos16, mu, te1, teb1, te2, teb2)


def _tor_reduce(tp, smat, bn=1024, rb=64):
    M, Dg = tp.shape
    MB = M // K
    grid = M // bn

    def kern(tr, sr, outr, str_):
        r = _dot(sr[...], tr[...])
        row = pl.program_id(0) * rb + lax.broadcasted_iota(
            jnp.int32, (rb, 1), 0).astype(jnp.float32)
        msk = jnp.where(row < float(NROT), 1.0, 0.0)
        outr[...] = r

        @pl.when(pl.program_id(0) == 0)
        def _():
            str_[...] = jnp.zeros_like(str_)

        str_[0:1, :] += jnp.sum(r * msk, axis=0, keepdims=True)
        str_[1:2, :] += jnp.sum(r * r * msk, axis=0, keepdims=True)

    return pl.pallas_call(
        kern, grid=(grid,),
        in_specs=[pl.BlockSpec((bn, Dg), lambda i: (i, 0)),
                  pl.BlockSpec((rb, bn), lambda i: (0, 0))],
        out_specs=[pl.BlockSpec((rb, Dg), lambda i: (i, 0)),
                   pl.BlockSpec((8, Dg), lambda i: (0, 0))],
        out_shape=[jax.ShapeDtypeStruct((MB, Dg), jnp.float32),
                   jax.ShapeDtypeStruct((8, Dg), jnp.float32)])(tp, smat)


def _tu_mlp(tu_bn, f1, f2row, bn=1024):
    M, Dg = tu_bn.shape
    grid = M // bn

    def kern(tr, f1r, f2r, outr):
        h = jnp.maximum(_dot(tr[...], f1r[...]), 0.0)
        t = jnp.sum(h * f2r[...], axis=1, keepdims=True)
        t = jnp.tanh(t) * jnp.pi + 1e-4
        outr[...] = jnp.broadcast_to(t, (bn, 128))

    return pl.pallas_call(
        kern, grid=(grid,),
        in_specs=[pl.BlockSpec((bn, Dg), lambda i: (i, 0)),
                  pl.BlockSpec((Dg, 128), lambda i: (0, 0)),
                  pl.BlockSpec((1, 128), lambda i: (0, 0))],
        out_specs=pl.BlockSpec((bn, 128), lambda i: (i, 0)),
        out_shape=jax.ShapeDtypeStruct((M, 128), jnp.float32))(
            tu_bn, f1, f2row)


def _sin_axis(g0, g1, tu_b):
    M = g0.shape[0]

    def kern(g0r, g1r, tur, outr):
        av = g1r[...] - g0r[...]
        n = jnp.sqrt(jnp.sum(av * av, axis=1, keepdims=True))
        u = av / (n + 1e-8)
        outr[...] = jnp.sin(tur[:, 0:1]) * u

    return pl.pallas_call(
        kern, grid=(1,),
        in_specs=[pl.BlockSpec((M, 16), lambda i: (0, 0)),
                  pl.BlockSpec((M, 16), lambda i: (0, 0)),
                  pl.BlockSpec((M, 128), lambda i: (0, 0))],
        out_specs=pl.BlockSpec((M, 16), lambda i: (0, 0)),
        out_shape=jax.ShapeDtypeStruct((M, 16), jnp.float32))(g0, g1, tu_b)


def _new_pos(pos, gs, bn=1000):
    M = pos.shape[0]
    grid = M // bn

    def kern(pr, gr, outr):
        outr[...] = pr[...] + gr[:, 0:3]

    return pl.pallas_call(
        kern, grid=(grid,),
        in_specs=[pl.BlockSpec((bn, 3), lambda i: (i, 0)),
                  pl.BlockSpec((bn, 16), lambda i: (i, 0))],
        out_specs=pl.BlockSpec((bn, 3), lambda i: (i, 0)),
        out_shape=jax.ShapeDtypeStruct((M, 3), jnp.float32))(pos, gs)


def _pad_rows(a, rows, value=0):
    return jnp.pad(a, ((0, rows - a.shape[0]), (0, 0)), constant_values=value)


def _pad_1d(a, n, value):
    return jnp.pad(a, (0, n - a.shape[0]), constant_values=value)


def _row(v):
    return v.reshape(1, -1)


def kernel(atom_features, atom_pos, bond_features, bond_sh, bond_edge_index,
           spatial_features, spatial_sh, spatial_edge_index, rotatable_bonds,
           batch, update_instructs, params):
    p = params
    f32 = jnp.float32

    bd = _pad_1d(bond_edge_index[0], EP, 0)
    bs = _pad_1d(bond_edge_index[1], EP, 0)
    sd = _pad_1d(spatial_edge_index[0], EP, 0)
    bs_sc = _pad_1d(bond_edge_index[1], EP, N).reshape(EP // 128, 128)
    ss_sc = _pad_1d(spatial_edge_index[1], EP, N).reshape(EP // 128, 128)

    bf = _pad_rows(bond_features, EP)
    bshp = jnp.pad(bond_sh, ((0, EP - E), (0, 16 - SD)))
    sf = _pad_rows(spatial_features, EP)
    sshp = jnp.pad(spatial_sh, ((0, EP - E), (0, 16 - SD)))

    zeros128 = jnp.zeros((128, D), f32)
    ones128 = jnp.ones((EP, D), f32)

    cb = _sc_scatter_add(ones128, bs_sc, zeros128)
    cs = _sc_scatter_add(ones128, ss_sc, zeros128)
    cb0, cb1 = cb[0, :N], cb[1, :N]
    cs0, cs1 = cs[0, :N], cs[1, :N]

    x = atom_features

    def tconv(x, ef, esh, dstI, srcI2d, mx, w1, b1, w2, b2, msh, bg, bb,
              c0, c1):
        xp = _mm(x, mx)
        g = _sc_gather(xp, dstI)
        mshp = jnp.pad(msh, ((0, 16 - SD), (0, 0)))
        tp = _edge_dense(g, ef, esh, w1, _row(b1), w2, _row(b2), mshp)
        sums = _sc_scatter_add(tp, srcI2d, zeros128)
        t, stats = _finish_a(sums[0, :N], sums[1, :N], c0, c1, x)
        return _finish_b(t, stats, _row(bg), _row(bb), float(N))

    for l in range(5):
        x = tconv(x, bf, bshp, bd, bs_sc,
                  p['b_Mx'][l], p['b_fc1_W'][l], p['b_fc1_b'][l],
                  p['b_fc2_W'][l], p['b_fc2_b'][l], p['b_Msh'][l],
                  p['b_bn_g'][l], p['b_bn_b'][l], cb0, cb1)
        a = jnp.pad(_mm(x, p['eu_lin_W'][l]), ((0, 0), (0, D - H)))
        gab = _sc_gather(a, jnp.concatenate([bd, bs]))
        ga = gab[:EP, :H]
        gb = gab[EP:, :H]
        w1 = p['eu_fc1_W'][l]
        bf = _edge_mlp(ga, gb, bf, w1[:H], w1[H:2 * H], w1[2 * H:],
                       _row(p['eu_fc1_b'][l]), p['eu_fc2_W'][l],
                       _row(p['eu_fc2_b'][l]), p['eu_fc3_W'][l],
                       _row(p['eu_fc3_b'][l]), _row(p['eu_ln_g'][l]),
                       _row(p['eu_ln_b'][l]))
        x = tconv(x, sf, sshp, sd, ss_sc,
                  p['s_Mx'][l], p['s_fc1_W'][l], p['s_fc1_b'][l],
                  p['s_fc2_W'][l], p['s_fc2_b'][l], p['s_Msh'][l],
                  p['s_bn_g'][l], p['s_bn_b'][l], cs0, cs1)

    pos_tbl = jnp.pad(atom_pos, ((0, NUP - N), (0, 128 - 3)))

    beT = jnp.pad(bond_edge_index.T, ((0, 0), (0, 126)))
    rotP = _pad_1d(rotatable_bonds, NROTP, 0)
    grb = _sc_gather(beT, rotP, chunk=64)
    rb0 = grb[:, 0]
    rb1 = grb[:, 1]
    g0 = _sc_gather(pos_tbl, rb0, chunk=64)[:, :16]
    g1p = _sc_gather(pos_tbl, rb1, chunk=64)[:, :16]

    bp_aug, bp_plain = _prep_bonds(g0, g1p)
    a_aug = _prep_atoms(pos_tbl[:10240, :16])
    asq_row = jnp.pad(a_aug[:, 3:4].T, ((0, 7), (0, 0)))
    nn = _topk(bp_aug, a_aug, asq_row)
    ti1 = nn.reshape(-1)

    gpos1 = _sc_gather(pos_tbl, ti1)[:, :16]
    gx1 = _sc_gather(x, ti1)
    xp_t = _mm(x, p['t_Mx'])
    gxp = _sc_gather(xp_t, ti1)

    mu = _row(jnp.linspace(0.0, DMAX, NRBF).astype(f32))
    ea, tesh = _tor_feat(gpos1, bp_plain, pos_tbl[:ETP // K, :16], mu,
                         p['te_fc1_W'], _row(p['te_fc1_b']),
                         p['te_fc2_W'], _row(p['te_fc2_b']))

    tw1 = p['t_fc1_W']
    tmsh = jnp.pad(p['t_Msh'], ((0, 128 - SD * 5), (0, 0)))
    tp_t = _edge_dense_tor(gxp, ea, gx1, x[:ETP // 256], tesh,
                           tw1[:H], tw1[H:H + NS], tw1[H + NS:],
                           _row(p['t_fc1_b']), p['t_fc2_W'],
                           _row(p['t_fc2_b']), tmsh)

    smat = (jnp.arange(1024) // K == jnp.arange(64)[:, None]).astype(f32) / K
    tu_pre, stats_t = _tor_reduce(tp_t, smat)
    tu_bn = _finish_b(tu_pre, stats_t, _row(p['t_bn_g']), _row(p['t_bn_b']),
                      float(NROT), bn=1024)
    f2row = _row(p['f2_W'][:, 0])
    tu_b = _tu_mlp(tu_bn, p['f1_W'], f2row)

    sv = jnp.pad(_sin_axis(g0, g1p, tu_b), ((0, 0), (0, 128 - 16)))
    uiP = _pad_1d(update_instructs, NUP, 0)
    gs = _sc_gather(sv, uiP)[:, :16]
    new_pos = _new_pos(atom_pos, gs[:N])

    return (new_pos, x, bf[:E], tu_b[:NROT, 0])
